# Initial kernel scaffold; baseline (speedup 1.0000x reference)
import functools

import jax
import jax.numpy as jnp
from jax import lax
from jax.experimental import pallas as pl
from jax.experimental.pallas import tpu as pltpu
from jax.experimental.pallas import tpu_sc as plsc

K = 64
N = 32768
ROWS = 128
NVEC = N // 16
GT_CAP = 224
GT_BUF = 320
HIST_EXIT = 160

_U = jnp.uint32


def _keys(v):
  kb = plsc.bitcast(v, jnp.uint32)
  sign = kb >> _U(31)
  flip = (_U(0) - sign) | _U(0x80000000)
  return kb ^ flip


def _scalar(x):
  return jnp.max(x)


def _sc_body(in_hbm, out_hbm, row_v, hist_v, gt_k, gt_i, eq_i, outst):
  c = lax.axis_index("c")
  s = lax.axis_index("s")
  wid = c * 16 + s
  iota = lax.iota(jnp.int32, 16)
  ones = jnp.ones((16,), jnp.int32)
  zeros_u = jnp.zeros((16,), jnp.uint32)
  zeros_i = jnp.zeros((16,), jnp.int32)

  def per_row(r, _):
    row = wid * 4 + r
    pltpu.sync_copy(in_hbm.at[row], row_v)

    def clr_gt(ci, _):
      gt_k[pl.ds(ci * 16, 16)] = zeros_u
      return 0
    lax.fori_loop(0, GT_BUF // 16, clr_gt, 0)

    def level_cond(carry):
      return jnp.logical_not(carry[5])

    def level_body(carry):
      level, pmask, pval, kneed, _, _ = carry
      is_last = level >= 2
      shift = jnp.where(is_last, 0, 20 - 12 * level).astype(jnp.uint32)
      bmask = jnp.where(is_last, _U(0xFF), _U(0xFFF))
      nb_vec = jnp.where(is_last, 16, 256)

      def clr(ci, _):
        hist_v[pl.ds(ci * 16, 16)] = zeros_i
        return 0
      lax.fori_loop(0, nb_vec, clr, 0)

      def hist_pass(i, _):
        ku = _keys(row_v[pl.ds(i * 16, 16)])
        part = (ku & pmask) == pval
        b = ((ku >> shift) & bmask).astype(jnp.int32)
        plsc.addupdate_scatter(hist_v, [b], ones, mask=part)
        return 0
      lax.fori_loop(0, NVEC, hist_pass, 0)

      def scan_cond(sc):
        return jnp.logical_and(sc[0] >= 0, jnp.logical_not(sc[5]))

      def scan_body(sc):
        vi, acc, B, SB, hB, done = sc
        vec = hist_v[pl.ds(vi * 16, 16)]
        tot = jnp.sum(vec)
        cum = jnp.cumsum(vec)
        suf = acc + tot - cum + vec
        m = suf >= kneed
        crossed = jnp.any(m)
        blane = jnp.max(jnp.where(m, iota, -1))
        sel = iota == blane
        nB = vi * 16 + blane
        nSB = jnp.max(jnp.where(sel, suf, 0))
        nhB = jnp.max(jnp.where(sel, vec, 0))
        B = jnp.where(crossed, nB, B)
        SB = jnp.where(crossed, nSB, SB)
        hB = jnp.where(crossed, nhB, hB)
        return (vi - 1, acc + tot, B, SB, hB, done | crossed)

      _, _, B, SB, histB, _ = lax.while_loop(
          scan_cond, scan_body,
          (nb_vec - 1, jnp.int32(0), jnp.int32(0), jnp.int32(0),
           jnp.int32(0), jnp.bool_(False)))

      T = pval | (B.astype(jnp.uint32) << shift)
      done = jnp.logical_or(histB <= HIST_EXIT, is_last)
      pmask = pmask | (bmask << shift)
      kneed = kneed - (SB - histB)
      return (level + 1, pmask, T, kneed, T, done)

    init = (jnp.int32(0), _U(0), _U(0), jnp.int32(K), _U(0), jnp.bool_(False))
    _, _, _, _, T, _ = lax.while_loop(level_cond, level_body, init)

    def collect(i, carry):
      cg, ce = carry
      ku = _keys(row_v[pl.ds(i * 16, 16)])
      mgt = jnp.logical_and(ku > T, cg < GT_CAP)
      meq = jnp.logical_and(ku == T, ce < 64)
      idxv = iota + i * 16
      plsc.store_compressed(gt_k.at[pl.ds(cg, 16)], ku, mask=mgt)
      plsc.store_compressed(gt_i.at[pl.ds(cg, 16)], idxv, mask=mgt)
      plsc.store_compressed(eq_i.at[pl.ds(ce, 16)], idxv, mask=meq)
      cg = cg + _scalar(plsc.all_reduce_population_count(mgt))
      ce = ce + _scalar(plsc.all_reduce_population_count(meq))
      return (cg, ce)

    cg, ce = lax.fori_loop(0, NVEC, collect, (jnp.int32(0), jnp.int32(0)))

    for a in range(5):
      lanepos = iota + a * 16
      gt_k[pl.ds(cg + a * 16, 16)] = jnp.where(lanepos < ce, T, _U(0))
      gt_i[pl.ds(cg + a * 16, 16)] = eq_i[pl.ds(a * 16, 16)]

    n_vec = (cg + 80 + 15) // 16

    def select(j, carry):
      vvec, ivec = carry

      def find(vi, fc):
        m, vb = fc
        vm = jnp.max(gt_k[pl.ds(vi * 16, 16)])
        upd = vm > m
        return (jnp.where(upd, vm, m), jnp.where(upd, vi, vb))

      m, vb = lax.fori_loop(0, n_vec, find, (_U(0), jnp.int32(0)))
      kv = gt_k[pl.ds(vb * 16, 16)]
      mm = kv == m
      lane = jnp.min(jnp.where(mm, iota, 16))
      sel = iota == lane
      isel = jnp.max(jnp.where(sel, gt_i[pl.ds(vb * 16, 16)], 0))
      gt_k[pl.ds(vb * 16, 16)] = jnp.where(sel, _U(0), kv)

      bits = m ^ ((~m >> _U(31)) | _U(0x80000000))
      val = lax.bitcast_convert_type(bits, jnp.float32)
      fidx = isel.astype(jnp.float32)
      g = j // 16
      l = j - g * 16
      vvec = jnp.where(iota == l, val, vvec)
      ivec = jnp.where(iota == l, fidx, ivec)

      @pl.when(l == 15)
      def _flush():
        outst[pl.ds(g * 16, 16)] = ivec
        outst[pl.ds(64 + g * 16, 16)] = vvec

      return (vvec, ivec)

    lax.fori_loop(0, K, select,
                  (jnp.zeros((16,), jnp.float32), jnp.zeros((16,), jnp.float32)))

    pltpu.sync_copy(outst, out_hbm.at[row])
    return 0

  lax.fori_loop(0, 4, per_row, 0)


_sc_topk = functools.partial(
    pl.kernel,
    out_type=jax.ShapeDtypeStruct((ROWS, 2 * K), jnp.float32),
    mesh=plsc.VectorSubcoreMesh(core_axis_name="c", subcore_axis_name="s"),
    scratch_types=[
        pltpu.VMEM((N,), jnp.float32),
        pltpu.VMEM((4096,), jnp.int32),
        pltpu.VMEM((GT_BUF,), jnp.uint32),
        pltpu.VMEM((GT_BUF,), jnp.int32),
        pltpu.VMEM((80,), jnp.int32),
        pltpu.VMEM((2 * K,), jnp.float32),
    ],
)(_sc_body)


def _norm_body(x_ref, o_ref):
  x = x_ref[...]
  val = x[:, K:]
  sc = jnp.sum(jnp.abs(val)) + 1e-6
  o_ref[:, :K] = x[:, :K]
  o_ref[:, K:] = val / sc


_norm = pl.pallas_call(
    _norm_body,
    out_shape=jax.ShapeDtypeStruct((ROWS, 2 * K), jnp.float32),
)


def kernel(inputs):
  return _norm(_sc_topk(inputs))


# speedup vs baseline: 1.6716x; 1.6716x over previous
import functools

import jax
import jax.numpy as jnp
from jax import lax
from jax.experimental import pallas as pl
from jax.experimental.pallas import tpu as pltpu
from jax.experimental.pallas import tpu_sc as plsc

K = 64
N = 32768
ROWS = 128
NVEC = N // 16
GT_BUF = 176
EQ_BUF = 80

_U = jnp.uint32
_LEVELS = ((20, 0xFFF, 256), (8, 0xFFF, 256), (0, 0xFF, 16))


def _keys(v):
  kb = plsc.bitcast(v, jnp.uint32)
  sign = kb >> _U(31)
  flip = (_U(0) - sign) | _U(0x80000000)
  return kb ^ flip


def _scalar(x):
  return jnp.max(x)


def _sc_body(in_hbm, out_hbm, row_v, hist_v, gt_k, gt_i, eq_i, outst):
  c = lax.axis_index("c")
  s = lax.axis_index("s")
  wid = c * 16 + s
  iota = lax.iota(jnp.int32, 16)
  ones = jnp.ones((16,), jnp.int32)
  zeros_u = jnp.zeros((16,), jnp.uint32)
  zeros_i = jnp.zeros((16,), jnp.int32)

  def per_row(r, _):
    row = wid * 4 + r
    pltpu.sync_copy(in_hbm.at[row], row_v)

    def clr_gt(ci, _):
      gt_k[pl.ds(ci * 16, 16)] = zeros_u
      return 0
    lax.fori_loop(0, GT_BUF // 16, clr_gt, 0)

    pmask = _U(0)
    pval = _U(0)
    kneed = jnp.int32(K)
    for shift_c, bmask_c, nb_vec in _LEVELS:
      shift = _U(shift_c)
      bmask = _U(bmask_c)

      def clr(ci, _):
        hist_v[pl.ds(ci * 16, 16)] = zeros_i
        return 0
      lax.fori_loop(0, nb_vec, clr, 0)

      def hist_pass(i, _, pmask=pmask, pval=pval, shift=shift, bmask=bmask):
        ku = _keys(row_v[pl.ds(i * 16, 16)])
        part = (ku & pmask) == pval
        b = ((ku >> shift) & bmask).astype(jnp.int32)
        plsc.addupdate_scatter(hist_v, [b], ones, mask=part)
        return 0
      lax.fori_loop(0, NVEC, hist_pass, 0)

      def scan_body(i, sc, nb_vec=nb_vec, kneed=kneed):
        acc, B, SB, hB, found = sc
        vi = nb_vec - 1 - i
        vec = hist_v[pl.ds(vi * 16, 16)]
        tot = jnp.sum(vec)
        cum = jnp.cumsum(vec)
        suf = acc + tot - cum + vec
        m = suf >= kneed
        crossed = jnp.logical_and(jnp.any(m), jnp.logical_not(found))
        blane = jnp.max(jnp.where(m, iota, -1))
        sel = iota == blane
        B = jnp.where(crossed, vi * 16 + blane, B)
        SB = jnp.where(crossed, jnp.max(jnp.where(sel, suf, 0)), SB)
        hB = jnp.where(crossed, jnp.max(jnp.where(sel, vec, 0)), hB)
        return (acc + tot, B, SB, hB, found | crossed)

      _, B, SB, histB, _ = lax.fori_loop(
          0, nb_vec, scan_body,
          (jnp.int32(0), jnp.int32(0), jnp.int32(0), jnp.int32(0),
           jnp.bool_(False)))

      pval = pval | (B.astype(jnp.uint32) << shift)
      pmask = pmask | (bmask << shift)
      kneed = kneed - (SB - histB)
    T = pval

    def collect(i, carry):
      cg, ce = carry
      ku = _keys(row_v[pl.ds(i * 16, 16)])
      mgt = jnp.logical_and(ku > T, cg < 80)
      meq = jnp.logical_and(ku == T, ce < 64)
      idxv = iota + i * 16
      plsc.store_compressed(gt_k.at[pl.ds(cg, 16)], ku, mask=mgt)
      plsc.store_compressed(gt_i.at[pl.ds(cg, 16)], idxv, mask=mgt)
      plsc.store_compressed(eq_i.at[pl.ds(ce, 16)], idxv, mask=meq)
      cg = cg + _scalar(plsc.all_reduce_population_count(mgt))
      ce = ce + _scalar(plsc.all_reduce_population_count(meq))
      return (cg, ce)

    cg, ce = lax.fori_loop(0, NVEC, collect, (jnp.int32(0), jnp.int32(0)))

    for a in range(EQ_BUF // 16):
      lanepos = iota + a * 16
      gt_k[pl.ds(cg + a * 16, 16)] = jnp.where(lanepos < ce, T, _U(0))
      gt_i[pl.ds(cg + a * 16, 16)] = eq_i[pl.ds(a * 16, 16)]

    n_vec = GT_BUF // 16

    def select(j, carry):
      vvec, ivec = carry

      def find(vi, fc):
        m, vb = fc
        vm = jnp.max(gt_k[pl.ds(vi * 16, 16)])
        upd = vm > m
        return (jnp.where(upd, vm, m), jnp.where(upd, vi, vb))

      m, vb = lax.fori_loop(0, n_vec, find, (_U(0), jnp.int32(0)))
      kv = gt_k[pl.ds(vb * 16, 16)]
      lane = jnp.min(jnp.where(kv == m, iota, 16))
      sel = iota == lane
      isel = jnp.max(jnp.where(sel, gt_i[pl.ds(vb * 16, 16)], 0))
      gt_k[pl.ds(vb * 16, 16)] = jnp.where(sel, _U(0), kv)

      bits = m ^ ((~m >> _U(31)) | _U(0x80000000))
      val = lax.bitcast_convert_type(bits, jnp.float32)
      fidx = isel.astype(jnp.float32)
      g = j // 16
      l = j - g * 16
      vvec = jnp.where(iota == l, val, vvec)
      ivec = jnp.where(iota == l, fidx, ivec)
      outst[pl.ds(g * 16, 16)] = ivec
      outst[pl.ds(64 + g * 16, 16)] = vvec
      return (vvec, ivec)

    lax.fori_loop(0, K, select,
                  (jnp.zeros((16,), jnp.float32), jnp.zeros((16,), jnp.float32)))

    pltpu.sync_copy(outst, out_hbm.at[row])
    return 0

  lax.fori_loop(0, 4, per_row, 0)


_sc_topk = functools.partial(
    pl.kernel,
    out_type=jax.ShapeDtypeStruct((ROWS, 2 * K), jnp.float32),
    mesh=plsc.VectorSubcoreMesh(core_axis_name="c", subcore_axis_name="s"),
    compiler_params=pltpu.CompilerParams(needs_layout_passes=False),
    scratch_types=[
        pltpu.VMEM((N,), jnp.float32),
        pltpu.VMEM((4096,), jnp.int32),
        pltpu.VMEM((GT_BUF,), jnp.uint32),
        pltpu.VMEM((GT_BUF,), jnp.int32),
        pltpu.VMEM((EQ_BUF,), jnp.int32),
        pltpu.VMEM((2 * K,), jnp.float32),
    ],
)(_sc_body)


def _norm_body(x_ref, o_ref):
  x = x_ref[...]
  val = x[:, K:]
  sc = jnp.sum(jnp.abs(val)) + 1e-6
  o_ref[:, :K] = x[:, :K]
  o_ref[:, K:] = val / sc


_norm = pl.pallas_call(
    _norm_body,
    out_shape=jax.ShapeDtypeStruct((ROWS, 2 * K), jnp.float32),
)


def kernel(inputs):
  return _norm(_sc_topk(inputs))


# speedup vs baseline: 1.8049x; 1.0798x over previous
import functools

import jax
import jax.numpy as jnp
from jax import lax
from jax.experimental import pallas as pl
from jax.experimental.pallas import tpu as pltpu
from jax.experimental.pallas import tpu_sc as plsc

K = 64
N = 32768
ROWS = 128
NVEC = N // 16
GT_BUF = 176
EQ_BUF = 80

_U = jnp.uint32
_LEVELS = ((20, 0xFFF, 256), (8, 0xFFF, 256), (0, 0xFF, 16))


def _keys(v):
  kb = plsc.bitcast(v, jnp.uint32)
  sign = kb >> _U(31)
  flip = (_U(0) - sign) | _U(0x80000000)
  return kb ^ flip


def _scalar(x):
  return x[0]


def _sc_body(in_hbm, out_hbm, row_v, hist_v, gt_k, gt_i, eq_i, outst):
  c = lax.axis_index("c")
  s = lax.axis_index("s")
  wid = c * 16 + s
  iota = lax.iota(jnp.int32, 16)
  ones = jnp.ones((16,), jnp.int32)
  zeros_u = jnp.zeros((16,), jnp.uint32)
  zeros_i = jnp.zeros((16,), jnp.int32)

  def per_row(r, _):
    row = wid * 4 + r
    pltpu.sync_copy(in_hbm.at[row], row_v)

    def clr_gt(ci, _):
      gt_k[pl.ds(ci * 16, 16)] = zeros_u
      return 0
    lax.fori_loop(0, GT_BUF // 16, clr_gt, 0)

    pmask = _U(0)
    pval = _U(0)
    kneed = jnp.int32(K)
    for shift_c, bmask_c, nb_vec in _LEVELS:
      shift = _U(shift_c)
      bmask = _U(bmask_c)

      def clr(ci, _):
        hist_v[pl.ds(ci * 16, 16)] = zeros_i
        return 0
      lax.fori_loop(0, nb_vec, clr, 0)

      def hist_pass(i, _, pmask=pmask, pval=pval, shift=shift, bmask=bmask):
        ku = _keys(row_v[pl.ds(i * 16, 16)])
        part = (ku & pmask) == pval
        b = ((ku >> shift) & bmask).astype(jnp.int32)
        plsc.addupdate_scatter(hist_v, [b], ones, mask=part)
        return 0
      lax.fori_loop(0, NVEC, hist_pass, 0)

      def scan_body(i, sc, nb_vec=nb_vec, kneed=kneed):
        acc, B, SB, hB, found = sc
        vi = nb_vec - 1 - i
        vec = hist_v[pl.ds(vi * 16, 16)]
        tot = jnp.sum(vec)
        cum = jnp.cumsum(vec)
        suf = acc + tot - cum + vec
        m = suf >= kneed
        crossed = jnp.logical_and(jnp.any(m), jnp.logical_not(found))
        blane = jnp.max(jnp.where(m, iota, -1))
        sel = iota == blane
        B = jnp.where(crossed, vi * 16 + blane, B)
        SB = jnp.where(crossed, jnp.max(jnp.where(sel, suf, 0)), SB)
        hB = jnp.where(crossed, jnp.max(jnp.where(sel, vec, 0)), hB)
        return (acc + tot, B, SB, hB, found | crossed)

      _, B, SB, histB, _ = lax.fori_loop(
          0, nb_vec, scan_body,
          (jnp.int32(0), jnp.int32(0), jnp.int32(0), jnp.int32(0),
           jnp.bool_(False)))

      pval = pval | (B.astype(jnp.uint32) << shift)
      pmask = pmask | (bmask << shift)
      kneed = kneed - (SB - histB)
    T = pval

    def collect(i, carry):
      cg, ce = carry
      ku = _keys(row_v[pl.ds(i * 16, 16)])
      mgt = jnp.logical_and(ku > T, cg < 80)
      meq = jnp.logical_and(ku == T, ce < 64)
      idxv = iota + i * 16
      plsc.store_compressed(gt_k.at[pl.ds(cg, 16)], ku, mask=mgt)
      plsc.store_compressed(gt_i.at[pl.ds(cg, 16)], idxv, mask=mgt)
      plsc.store_compressed(eq_i.at[pl.ds(ce, 16)], idxv, mask=meq)
      cg = cg + _scalar(plsc.all_reduce_population_count(mgt))
      ce = ce + _scalar(plsc.all_reduce_population_count(meq))
      return (cg, ce)

    cg, ce = lax.fori_loop(0, NVEC, collect, (jnp.int32(0), jnp.int32(0)))

    for a in range(EQ_BUF // 16):
      lanepos = iota + a * 16
      gt_k[pl.ds(cg + a * 16, 16)] = jnp.where(lanepos < ce, T, _U(0))
      gt_i[pl.ds(cg + a * 16, 16)] = eq_i[pl.ds(a * 16, 16)]

    n_vec = GT_BUF // 16

    def select(j, carry):
      vvec, ivec = carry

      def find(vi, fc):
        m, vb = fc
        vm = jnp.max(gt_k[pl.ds(vi * 16, 16)])
        upd = vm > m
        return (jnp.where(upd, vm, m), jnp.where(upd, vi, vb))

      m, vb = lax.fori_loop(0, n_vec, find, (_U(0), jnp.int32(0)))
      kv = gt_k[pl.ds(vb * 16, 16)]
      lane = jnp.min(jnp.where(kv == m, iota, 16))
      sel = iota == lane
      isel = jnp.max(jnp.where(sel, gt_i[pl.ds(vb * 16, 16)], 0))
      gt_k[pl.ds(vb * 16, 16)] = jnp.where(sel, _U(0), kv)

      bits = m ^ ((~m >> _U(31)) | _U(0x80000000))
      val = lax.bitcast_convert_type(bits, jnp.float32)
      fidx = isel.astype(jnp.float32)
      g = j // 16
      l = j - g * 16
      vvec = jnp.where(iota == l, val, vvec)
      ivec = jnp.where(iota == l, fidx, ivec)
      outst[pl.ds(g * 16, 16)] = ivec
      outst[pl.ds(64 + g * 16, 16)] = vvec
      return (vvec, ivec)

    lax.fori_loop(0, K, select,
                  (jnp.zeros((16,), jnp.float32), jnp.zeros((16,), jnp.float32)))

    pltpu.sync_copy(outst, out_hbm.at[row])
    return 0

  lax.fori_loop(0, 4, per_row, 0)


_sc_topk = functools.partial(
    pl.kernel,
    out_type=jax.ShapeDtypeStruct((ROWS, 2 * K), jnp.float32),
    mesh=plsc.VectorSubcoreMesh(core_axis_name="c", subcore_axis_name="s"),
    compiler_params=pltpu.CompilerParams(needs_layout_passes=False),
    scratch_types=[
        pltpu.VMEM((N,), jnp.float32),
        pltpu.VMEM((4096,), jnp.int32),
        pltpu.VMEM((GT_BUF,), jnp.uint32),
        pltpu.VMEM((GT_BUF,), jnp.int32),
        pltpu.VMEM((EQ_BUF,), jnp.int32),
        pltpu.VMEM((2 * K,), jnp.float32),
    ],
)(_sc_body)


def _norm_body(x_ref, o_ref):
  x = x_ref[...]
  val = x[:, K:]
  sc = jnp.sum(jnp.abs(val)) + 1e-6
  o_ref[:, :K] = x[:, :K]
  o_ref[:, K:] = val / sc


_norm = pl.pallas_call(
    _norm_body,
    out_shape=jax.ShapeDtypeStruct((ROWS, 2 * K), jnp.float32),
)


def kernel(inputs):
  return _norm(_sc_topk(inputs))


# speedup vs baseline: 4.1466x; 2.2974x over previous
import functools

import jax
import jax.numpy as jnp
from jax import lax
from jax.experimental import pallas as pl
from jax.experimental.pallas import tpu as pltpu
from jax.experimental.pallas import tpu_sc as plsc

K = 64
N = 32768
ROWS = 128
NVEC = N // 16
GT_BUF = 256
EQ_BUF = 80
GT_CAP = 160
HIST_EXIT = 96

_U = jnp.uint32


def _keys(v):
  kb = plsc.bitcast(v, jnp.uint32)
  sign = kb >> _U(31)
  flip = (_U(0) - sign) | _U(0x80000000)
  return kb ^ flip


def _scalar(x):
  return x[0]


def _sc_body(in_hbm, out_hbm, row_v, hist_v, gt_k, gt_i, eq_i, out_row):
  c = lax.axis_index("c")
  s = lax.axis_index("s")
  wid = c * 16 + s
  iota = lax.iota(jnp.int32, 16)
  ones = jnp.ones((16,), jnp.int32)
  zeros_u = jnp.zeros((16,), jnp.uint32)
  zeros_i = jnp.zeros((16,), jnp.int32)
  lane0 = iota == 0

  def scan_hist(nb_vec, kneed):
    init = (jnp.int32(0), jnp.int32(0), jnp.int32(0), jnp.int32(0),
            jnp.bool_(False))

    @pl.loop(0, nb_vec, init_carry=init,
             unroll=4 if isinstance(nb_vec, int) else None)
    def scan_res(i, sc):
      acc, B, SB, hB, found = sc
      vi = nb_vec - 1 - i
      vec = hist_v[pl.ds(vi * 16, 16)]
      cum = jnp.cumsum(vec)
      tot = cum[15]
      suf = acc + tot - cum + vec
      m = suf >= kneed
      crossed = jnp.logical_and(jnp.any(m), jnp.logical_not(found))
      blane = jnp.max(jnp.where(m, iota, -1))
      sel = iota == blane
      B = jnp.where(crossed, vi * 16 + blane, B)
      SB = jnp.where(crossed, jnp.max(jnp.where(sel, suf, 0)), SB)
      hB = jnp.where(crossed, jnp.max(jnp.where(sel, vec, 0)), hB)
      return (acc + tot, B, SB, hB, found | crossed)

    _, B, SB, histB, _ = scan_res
    return B, SB, histB

  def per_row(r, _):
    row = wid * 4 + r
    pltpu.sync_copy(in_hbm.at[row], row_v)

    def clr(ci, _):
      hist_v[pl.ds(ci * 16, 16)] = zeros_i
      return 0
    lax.fori_loop(0, 256, clr, 0)
    for ci in range(GT_BUF // 16):
      gt_k[pl.ds(ci * 16, 16)] = zeros_u

    @plsc.parallel_loop(0, N, step=16, unroll=8)
    def _l0(i):
      ku = _keys(row_v[pl.ds(i, 16)])
      b = (ku >> _U(20)).astype(jnp.int32)
      plsc.addupdate_scatter(hist_v, [b], ones)

    B, SB, histB = scan_hist(256, jnp.int32(K))
    T0 = B.astype(jnp.uint32) << _U(20)

    refine = histB > HIST_EXIT
    kneed0 = jnp.int32(K) - (SB - histB)
    init = (_U(0xFFF00000), T0, kneed0, T0)

    @pl.loop(0, jnp.where(refine, 2, 0), init_carry=init)
    def ref_res(l, cr):
      pmask, pval, kneed, _ = cr
      is1 = l == 0
      shift = jnp.where(is1, _U(8), _U(0))
      bmask = jnp.where(is1, _U(0xFFF), _U(0xFF))
      nb_vec = jnp.where(is1, 256, 16)

      @pl.loop(0, nb_vec)
      def _c(ci):
        hist_v[pl.ds(ci * 16, 16)] = zeros_i

      def hp(i, _):
        ku = _keys(row_v[pl.ds(i * 16, 16)])
        part = (ku & pmask) == pval
        b = ((ku >> shift) & bmask).astype(jnp.int32)
        plsc.addupdate_scatter(hist_v, [b], ones, mask=part)
        return 0
      lax.fori_loop(0, NVEC, hp, 0)

      B2, SB2, histB2 = scan_hist(nb_vec, kneed)
      T2 = pval | (B2.astype(jnp.uint32) << shift)
      return (pmask | (bmask << shift), T2, kneed - (SB2 - histB2), T2)

    T = ref_res[3]

    @plsc.parallel_loop(0, N, step=16, unroll=4,
                        carry=(jnp.int32(0), jnp.int32(0)))
    def counts(i, carry):
      cg, ce = carry
      ku = _keys(row_v[pl.ds(i, 16)])
      mgt = jnp.logical_and(ku > T, cg < GT_CAP)
      meq = jnp.logical_and(ku == T, ce < 64)
      idxv = iota + i
      plsc.store_compressed(gt_k.at[pl.ds(cg, 16)], ku, mask=mgt)
      plsc.store_compressed(gt_i.at[pl.ds(cg, 16)], idxv, mask=mgt)
      plsc.store_compressed(eq_i.at[pl.ds(ce, 16)], idxv, mask=meq)
      cg = cg + _scalar(plsc.all_reduce_population_count(mgt))
      ce = ce + _scalar(plsc.all_reduce_population_count(meq))
      return (cg, ce)

    cg, ce = counts

    for a in range(EQ_BUF // 16):
      lanepos = iota + a * 16
      gt_k[pl.ds(cg + a * 16, 16)] = jnp.where(lanepos < ce, T, _U(0))
      gt_i[pl.ds(cg + a * 16, 16)] = eq_i[pl.ds(a * 16, 16)]

    maxv = jnp.zeros((16,), jnp.uint32)
    for vi in range(GT_BUF // 16):
      mvi = jnp.max(gt_k[pl.ds(vi * 16, 16)])
      maxv = jnp.where(iota == vi, mvi, maxv)

    def select(j, maxv):
      m = jnp.max(maxv)
      vb = _scalar(plsc.all_reduce_ffs(maxv == m))
      kv = gt_k[pl.ds(vb * 16, 16)]
      lane = _scalar(plsc.all_reduce_ffs(kv == m))
      pos = vb * 16 + lane
      isel = _scalar(plsc.load_gather(gt_i, [jnp.full((16,), pos, jnp.int32)]))
      kv2 = jnp.where(iota == lane, _U(0), kv)
      gt_k[pl.ds(vb * 16, 16)] = kv2
      maxv = jnp.where(iota == vb, jnp.max(kv2), maxv)

      bits = m ^ ((~m >> _U(31)) | _U(0x80000000))
      val = lax.bitcast_convert_type(bits, jnp.float32)
      fidx = isel.astype(jnp.float32)
      plsc.store_scatter(out_row, [jnp.full((16,), j, jnp.int32)],
                         jnp.full((16,), fidx, jnp.float32), mask=lane0)
      plsc.store_scatter(out_row, [jnp.full((16,), K + j, jnp.int32)],
                         jnp.full((16,), val, jnp.float32), mask=lane0)
      return maxv

    lax.fori_loop(0, K, select, maxv)

    pltpu.sync_copy(out_row, out_hbm.at[row])
    return 0

  lax.fori_loop(0, 4, per_row, 0)


_sc_topk = functools.partial(
    pl.kernel,
    out_type=jax.ShapeDtypeStruct((ROWS, 2 * K), jnp.float32),
    mesh=plsc.VectorSubcoreMesh(core_axis_name="c", subcore_axis_name="s"),
    compiler_params=pltpu.CompilerParams(needs_layout_passes=False),
    scratch_types=[
        pltpu.VMEM((N,), jnp.float32),
        pltpu.VMEM((4096,), jnp.int32),
        pltpu.VMEM((GT_BUF,), jnp.uint32),
        pltpu.VMEM((GT_BUF,), jnp.int32),
        pltpu.VMEM((EQ_BUF,), jnp.int32),
        pltpu.VMEM((2 * K,), jnp.float32),
    ],
)(_sc_body)


def _norm_body(x_ref, o_ref):
  x = x_ref[...]
  val = x[:, K:]
  sc = jnp.sum(jnp.abs(val)) + 1e-6
  o_ref[:, :K] = x[:, :K]
  o_ref[:, K:] = val / sc


_norm = pl.pallas_call(
    _norm_body,
    out_shape=jax.ShapeDtypeStruct((ROWS, 2 * K), jnp.float32),
)


def kernel(inputs):
  return _norm(_sc_topk(inputs))


# speedup vs baseline: 10.8770x; 2.6231x over previous
import functools

import jax
import jax.numpy as jnp
from jax import lax
from jax.experimental import pallas as pl
from jax.experimental.pallas import tpu as pltpu
from jax.experimental.pallas import tpu_sc as plsc

K = 64
N = 32768
ROWS = 128
NVEC = N // 16
GT_BUF = 256
EQ_BUF = 80
GT_CAP = 160
HIST_EXIT = 96

_U = jnp.uint32


def _keys(v):
  kb = plsc.bitcast(v, jnp.uint32)
  sign = kb >> _U(31)
  flip = (_U(0) - sign) | _U(0x80000000)
  return kb ^ flip


def _scalar(x):
  return x[0]


def _sc_body(in_hbm, out_hbm, row_v, hist_v, gt_k, gt_i, eq_i, outst):
  c = lax.axis_index("c")
  s = lax.axis_index("s")
  wid = c * 16 + s
  iota = lax.iota(jnp.int32, 16)
  ones = jnp.ones((16,), jnp.int32)
  zeros_u = jnp.zeros((16,), jnp.uint32)
  zeros_i = jnp.zeros((16,), jnp.int32)

  def scan_hist(nb_vec, kneed):
    static = isinstance(nb_vec, int)
    init = (jnp.int32(0), jnp.int32(0), jnp.int32(0), jnp.bool_(False))

    @pl.loop(0, nb_vec, init_carry=init, unroll=8 if static else None)
    def scan_res(i, sc):
      acc, vvi, vacc, found = sc
      vi = nb_vec - 1 - i
      cum = jnp.cumsum(hist_v[pl.ds(vi * 16, 16)])
      acc2 = acc + cum[15]
      crossed = jnp.logical_and(acc2 >= kneed, jnp.logical_not(found))
      vvi = jnp.where(crossed, vi, vvi)
      vacc = jnp.where(crossed, acc, vacc)
      return (acc2, vvi, vacc, found | crossed)

    _, vvi, vacc, _ = scan_res
    vec = hist_v[pl.ds(vvi * 16, 16)]
    cum = jnp.cumsum(vec)
    suf = vacc + cum[15] - cum + vec
    m = suf >= kneed
    blane = jnp.max(jnp.where(m, iota, -1))
    sel = iota == blane
    SB = jnp.max(jnp.where(sel, suf, 0))
    histB = jnp.max(jnp.where(sel, vec, 0))
    return vvi * 16 + blane, SB, histB

  def per_row(r, _):
    row = wid * 4 + r
    pltpu.sync_copy(in_hbm.at[row], row_v)

    for ci in range(GT_BUF // 16):
      gt_k[pl.ds(ci * 16, 16)] = zeros_u

    @plsc.parallel_loop(0, 4096, step=16, unroll=8)
    def _hc(i):
      hist_v[pl.ds(i, 16)] = zeros_i

    @plsc.parallel_loop(0, N, step=16, unroll=8)
    def _l0(i):
      ku = _keys(row_v[pl.ds(i, 16)])
      b = (ku >> _U(20)).astype(jnp.int32)
      plsc.addupdate_scatter(hist_v, [b], ones)

    B, SB, histB = scan_hist(256, jnp.int32(K))
    T0 = B.astype(jnp.uint32) << _U(20)

    refine = histB > HIST_EXIT
    kneed0 = jnp.int32(K) - (SB - histB)
    init = (_U(0xFFF00000), T0, kneed0, T0)

    @pl.loop(0, jnp.where(refine, 2, 0), init_carry=init)
    def ref_res(l, cr):
      pmask, pval, kneed, _ = cr
      is1 = l == 0
      shift = jnp.where(is1, _U(8), _U(0))
      bmask = jnp.where(is1, _U(0xFFF), _U(0xFF))
      nb_vec = jnp.where(is1, 256, 16)

      @pl.loop(0, nb_vec)
      def _c(ci):
        hist_v[pl.ds(ci * 16, 16)] = zeros_i

      def hp(i, _):
        ku = _keys(row_v[pl.ds(i * 16, 16)])
        part = (ku & pmask) == pval
        b = ((ku >> shift) & bmask).astype(jnp.int32)
        plsc.addupdate_scatter(hist_v, [b], ones, mask=part)
        return 0
      lax.fori_loop(0, NVEC, hp, 0)

      B2, SB2, histB2 = scan_hist(nb_vec, kneed)
      T2 = pval | (B2.astype(jnp.uint32) << shift)
      return (pmask | (bmask << shift), T2, kneed - (SB2 - histB2), T2)

    T = ref_res[3]

    @plsc.parallel_loop(0, N, step=16, unroll=8,
                        carry=jnp.zeros((16,), jnp.int32))
    def cgv(i, cgv):
      ku = _keys(row_v[pl.ds(i, 16)])
      mgt = jnp.logical_and(ku > T, cgv < GT_CAP)
      cs = _scalar(cgv)
      plsc.store_compressed(gt_k.at[pl.ds(cs, 16)], ku, mask=mgt)
      plsc.store_compressed(gt_i.at[pl.ds(cs, 16)], iota + i, mask=mgt)
      return cgv + plsc.all_reduce_population_count(mgt)

    cg = _scalar(cgv)

    @pl.loop(0, jnp.where(cg < K, 1, 0), init_carry=jnp.int32(0))
    def ce(_t, _ce):
      def eqp(i, cev):
        ku = _keys(row_v[pl.ds(i * 16, 16)])
        meq = jnp.logical_and(ku == T, cev < K)
        plsc.store_compressed(eq_i.at[pl.ds(_scalar(cev), 16)],
                              iota + i * 16, mask=meq)
        return cev + plsc.all_reduce_population_count(meq)
      return _scalar(lax.fori_loop(0, NVEC, eqp, zeros_i))

    for a in range(EQ_BUF // 16):
      lanepos = iota + a * 16
      gt_k[pl.ds(cg + a * 16, 16)] = jnp.where(lanepos < ce, T, _U(0))
      gt_i[pl.ds(cg + a * 16, 16)] = eq_i[pl.ds(a * 16, 16)]

    maxv = jnp.zeros((16,), jnp.uint32)
    for vi in range(GT_BUF // 16):
      maxv = jnp.where(iota == vi, jnp.max(gt_k[pl.ds(vi * 16, 16)]), maxv)

    def select(j, carry):
      maxv, vvec, ivec = carry
      m = jnp.max(maxv)
      vb = _scalar(plsc.all_reduce_ffs(maxv == m))
      kv = gt_k[pl.ds(vb * 16, 16)]
      lane = _scalar(plsc.all_reduce_ffs(kv == m))
      pos = vb * 16 + lane
      isel = _scalar(plsc.load_gather(gt_i, [jnp.full((16,), pos, jnp.int32)]))
      kv2 = jnp.where(iota == lane, _U(0), kv)
      gt_k[pl.ds(vb * 16, 16)] = kv2
      maxv = jnp.where(iota == vb, jnp.max(kv2), maxv)

      bits = m ^ ((~m >> _U(31)) | _U(0x80000000))
      val = lax.bitcast_convert_type(bits, jnp.float32)
      fidx = isel.astype(jnp.float32)
      l = j & 15
      base = j - l
      vvec = jnp.where(iota == l, val, vvec)
      ivec = jnp.where(iota == l, fidx, ivec)
      outst[pl.ds(base, 16)] = ivec
      outst[pl.ds(K + base, 16)] = vvec
      return (maxv, vvec, ivec)

    zf = jnp.zeros((16,), jnp.float32)
    lax.fori_loop(0, K, select, (maxv, zf, zf))

    pltpu.sync_copy(outst, out_hbm.at[row])
    return 0

  lax.fori_loop(0, 4, per_row, 0)


_sc_topk = functools.partial(
    pl.kernel,
    out_type=jax.ShapeDtypeStruct((ROWS, 2 * K), jnp.float32),
    mesh=plsc.VectorSubcoreMesh(core_axis_name="c", subcore_axis_name="s"),
    compiler_params=pltpu.CompilerParams(needs_layout_passes=False),
    scratch_types=[
        pltpu.VMEM((N,), jnp.float32),
        pltpu.VMEM((4096,), jnp.int32),
        pltpu.VMEM((GT_BUF,), jnp.uint32),
        pltpu.VMEM((GT_BUF,), jnp.int32),
        pltpu.VMEM((EQ_BUF,), jnp.int32),
        pltpu.VMEM((2 * K,), jnp.float32),
    ],
)(_sc_body)


def _norm_body(x_ref, o_ref):
  x = x_ref[...]
  val = x[:, K:]
  sc = jnp.sum(jnp.abs(val)) + 1e-6
  o_ref[:, :K] = x[:, :K]
  o_ref[:, K:] = val / sc


_norm = pl.pallas_call(
    _norm_body,
    out_shape=jax.ShapeDtypeStruct((ROWS, 2 * K), jnp.float32),
)


def kernel(inputs):
  return _norm(_sc_topk(inputs))
